# Initial kernel scaffold; baseline (speedup 1.0000x reference)
import jax
import jax.numpy as jnp
from jax.experimental import pallas as pl


def kernel(x, edge_index, e, xbatch, params):
    raise NotImplementedError("write your pallas kernel here")


# speedup vs baseline: 2.2492x; 2.2492x over previous
import functools

import jax
import jax.numpy as jnp
from jax import lax
from jax.experimental import pallas as pl
from jax.experimental.pallas import tpu as pltpu
from jax.experimental.pallas import tpu_sc as plsc

N = 10000
E = 160000
NODE_IN = 16
EDGE_IN = 19
LEAK = 0.1
DIMS = [(16, 32), (32, 32), (32, 32)]

NC = 2
NS = 16
NW = NC * NS
CHUNK = 128
NCH = 40
EPW = NCH * CHUNK
E_PAD = NW * EPW
PAD = E_PAD - E
N_ACC = N + PAD
RPS = N // NS

ETILE = 1024


def _leaky(v):
    return jnp.where(v >= 0, v, LEAK * v)


def _estats_body(e_ref, o_ref):
    i = pl.program_id(0)
    blk = e_ref[...]
    s = jnp.sum(blk, axis=0, keepdims=True)
    q = jnp.sum(blk * blk, axis=0, keepdims=True)
    part = jnp.concatenate([s, q], axis=0)

    @pl.when(i == 0)
    def _():
        o_ref[...] = part

    @pl.when(i != 0)
    def _():
        o_ref[...] += part


def _edge_stats(e_pad):
    rows = 1280
    grid = E_PAD // rows
    return pl.pallas_call(
        _estats_body,
        grid=(grid,),
        in_specs=[pl.BlockSpec((rows, EDGE_IN), lambda i: (i, 0))],
        out_specs=pl.BlockSpec((2, EDGE_IN), lambda i: (0, 0)),
        out_shape=jax.ShapeDtypeStruct((2, EDGE_IN), jnp.float32),
    )(e_pad)


def _xstats_body(x_ref, o_ref):
    x = x_ref[...]
    mu = jnp.mean(x, axis=0, keepdims=True)
    var = jnp.mean((x - mu) ** 2, axis=0, keepdims=True)
    o_ref[...] = jnp.concatenate([mu, var], axis=0)


def _node_stats(x):
    return pl.pallas_call(
        _xstats_body,
        out_shape=jax.ShapeDtypeStruct((2, NODE_IN), jnp.float32),
    )(x)


def _msg_body(e_ref, xs_ref, w1_ref, b1_ref, w2_ref, b2_ref, r_ref, s_ref,
              sx_ref, tx_ref, o_ref):
    xsn = xs_ref[...] * sx_ref[...] + tx_ref[...]
    h = _leaky(jnp.dot(e_ref[...], w1_ref[...],
                       preferred_element_type=jnp.float32) + b1_ref[...])
    w = _leaky(jnp.dot(h, w2_ref[...],
                       preferred_element_type=jnp.float32) + b2_ref[...])
    xsb = jnp.dot(xsn, r_ref[...], preferred_element_type=jnp.float32)
    o_ref[...] = jnp.dot(xsb * w, s_ref[...],
                         preferred_element_type=jnp.float32)


def _msg(e_pad, xs, w1, b1, w2, b2, rmat, smat, sx, tx, nin, nout):
    grid = E_PAD // ETILE
    full = lambda a: pl.BlockSpec(a.shape, lambda i: tuple(0 for _ in a.shape))
    return pl.pallas_call(
        _msg_body,
        grid=(grid,),
        in_specs=[
            pl.BlockSpec((ETILE, EDGE_IN), lambda i: (i, 0)),
            pl.BlockSpec((ETILE, nin), lambda i: (i, 0)),
            full(w1), full(b1), full(w2), full(b2), full(rmat), full(smat),
            full(sx), full(tx),
        ],
        out_specs=pl.BlockSpec((ETILE, nout), lambda i: (i, 0)),
        out_shape=jax.ShapeDtypeStruct((E_PAD, nout), jnp.float32),
        compiler_params=pltpu.CompilerParams(
            dimension_semantics=("arbitrary",)),
    )(e_pad, xs, w1, b1, w2, b2, rmat, smat, sx, tx)


def _update_body(p_ref, x_ref, rt_ref, bs_ref, o_ref):
    o_ref[...] = (p_ref[0] + p_ref[1]
                  + jnp.dot(x_ref[...], rt_ref[...],
                            preferred_element_type=jnp.float32)
                  + bs_ref[...])


def _update(part, x_cur, root, bias):
    return pl.pallas_call(
        _update_body,
        out_shape=jax.ShapeDtypeStruct((N, root.shape[1]), jnp.float32),
    )(part, x_cur, root, bias)


def _emlp_body(xs_ref, xd_ref, e_ref, a_ref, b_ref, c_ref, b1_ref,
               w2_ref, b2_ref, o_ref):
    h = (jnp.dot(xs_ref[...], a_ref[...], preferred_element_type=jnp.float32)
         + jnp.dot(xd_ref[...], b_ref[...], preferred_element_type=jnp.float32)
         + jnp.dot(e_ref[...], c_ref[...], preferred_element_type=jnp.float32)
         + b1_ref[...])
    h = _leaky(h)
    o_ref[...] = jnp.dot(h, w2_ref[...],
                         preferred_element_type=jnp.float32) + b2_ref[...]


def _emlp(xs, xd, e_pad, a, b, c, b1, w2p, b2p):
    grid = E_PAD // ETILE
    full = lambda arr: pl.BlockSpec(arr.shape,
                                    lambda i: tuple(0 for _ in arr.shape))
    return pl.pallas_call(
        _emlp_body,
        grid=(grid,),
        in_specs=[
            pl.BlockSpec((ETILE, 32), lambda i: (i, 0)),
            pl.BlockSpec((ETILE, 32), lambda i: (i, 0)),
            pl.BlockSpec((ETILE, EDGE_IN), lambda i: (i, 0)),
            full(a), full(b), full(c), full(b1), full(w2p), full(b2p),
        ],
        out_specs=pl.BlockSpec((ETILE, 32), lambda i: (i, 0)),
        out_shape=jax.ShapeDtypeStruct((E_PAD, 32), jnp.float32),
        compiler_params=pltpu.CompilerParams(
            dimension_semantics=("arbitrary",)),
    )(xs, xd, e_pad, a, b, c, b1, w2p, b2p)


def _head_body(x_ref, g_ref, w1a_ref, w1b_ref, b1_ref, w2_ref, b2_ref, o_ref):
    g = g_ref[0] + g_ref[1]
    h = _leaky(jnp.dot(x_ref[...], w1a_ref[...],
                       preferred_element_type=jnp.float32)
               + jnp.dot(g, w1b_ref[...], preferred_element_type=jnp.float32)
               + b1_ref[...])
    v = jnp.dot(h, w2_ref[...], preferred_element_type=jnp.float32) + b2_ref[...]
    m = jnp.max(v, axis=1, keepdims=True)
    lse = m + jnp.log(jnp.sum(jnp.exp(v - m), axis=1, keepdims=True))
    o_ref[...] = v - lse


def _head(x_cur, gpart, w1a, w1b, b1, w2, b2):
    return pl.pallas_call(
        _head_body,
        out_shape=jax.ShapeDtypeStruct((N, 2), jnp.float32),
    )(x_cur, gpart, w1a, w1b, b1, w2, b2)


def _sc_gather_body(x_hbm, idx_hbm, out_hbm, idxv, rowbuf, sem):
    cid = lax.axis_index("c")
    sid = lax.axis_index("s")
    wid = cid * NS + sid
    pltpu.sync_copy(idx_hbm.at[wid], idxv)

    def body(j, carry):
        base = wid * EPW + j * CHUNK
        pltpu.async_copy(x_hbm.at[idxv.at[j]], rowbuf, sem).wait()
        pltpu.sync_copy(rowbuf, out_hbm.at[pl.ds(base, CHUNK)])
        return carry

    lax.fori_loop(0, NCH, body, 0)


@functools.lru_cache(maxsize=None)
def _make_gather(d):
    mesh = plsc.VectorSubcoreMesh(core_axis_name="c", subcore_axis_name="s")
    return pl.kernel(
        _sc_gather_body,
        out_type=jax.ShapeDtypeStruct((E_PAD, d), jnp.float32),
        mesh=mesh,
        scratch_types=[
            pltpu.VMEM((NCH, CHUNK), jnp.int32),
            pltpu.VMEM((CHUNK, d), jnp.float32),
            pltpu.SemaphoreType.DMA,
        ],
        compiler_params=pltpu.CompilerParams(use_tc_tiling_on_sc=False),
    )


def _sc_scatter_body(msg_hbm, idx_hbm, z_hbm, out_hbm, idxv, mbuf, obuf, accum):
    cid = lax.axis_index("c")
    sid = lax.axis_index("s")
    wid = cid * NS + sid
    pltpu.sync_copy(z_hbm.at[pl.ds(sid * RPS, RPS)], obuf)
    pltpu.sync_copy(obuf, accum.at[pl.ds(sid * RPS, RPS)])
    pltpu.sync_copy(idx_hbm.at[wid], idxv)
    plsc.subcore_barrier()

    def body(j, carry):
        base = wid * EPW + j * CHUNK
        pltpu.sync_copy(msg_hbm.at[pl.ds(base, CHUNK)], mbuf)
        pltpu.sync_copy(mbuf, accum.at[idxv.at[j]], add=True)
        return carry

    lax.fori_loop(0, NCH, body, 0)
    plsc.subcore_barrier()
    pltpu.sync_copy(accum.at[pl.ds(sid * RPS, RPS)], obuf)
    pltpu.sync_copy(obuf, out_hbm.at[cid, pl.ds(sid * RPS, RPS)])


@functools.lru_cache(maxsize=None)
def _make_scatter():
    mesh = plsc.VectorSubcoreMesh(core_axis_name="c", subcore_axis_name="s")
    return pl.kernel(
        _sc_scatter_body,
        out_type=jax.ShapeDtypeStruct((NC, N, 32), jnp.float32),
        mesh=mesh,
        scratch_types=[
            pltpu.VMEM((NCH, CHUNK), jnp.int32),
            pltpu.VMEM((CHUNK, 32), jnp.float32),
            pltpu.VMEM((RPS, 32), jnp.float32),
            pltpu.VMEM_SHARED((N_ACC, 32), jnp.float32),
        ],
        compiler_params=pltpu.CompilerParams(use_tc_tiling_on_sc=False),
    )


def kernel(x, edge_index, e, xbatch, params):
    f32 = jnp.float32
    x = x.astype(f32)
    e = e.astype(f32)
    src = edge_index[0]
    dst = edge_index[1]

    pad_fill = (jnp.arange(PAD, dtype=jnp.int32) * 13) % N
    src3 = jnp.concatenate([src, pad_fill]).reshape(NW, NCH, CHUNK)
    dst3g = jnp.concatenate([dst, pad_fill]).reshape(NW, NCH, CHUNK)
    dst3s = jnp.concatenate(
        [dst, N + jnp.arange(PAD, dtype=jnp.int32)]).reshape(NW, NCH, CHUNK)

    e_pad = jnp.zeros((E_PAD, EDGE_IN), f32).at[:E, :].set(e)
    zeros_n32 = jnp.zeros((N, 32), f32)

    est = _edge_stats(e_pad)
    mu_e = est[0] / E
    var_e = est[1] / E - mu_e * mu_e
    s_e = params['bn_edge_g'] / jnp.sqrt(var_e + 1e-5)
    t_e = params['bn_edge_b'] - mu_e * s_e

    xst = _node_stats(x)
    s_x = params['bn_node_g'] / jnp.sqrt(xst[1] + 1e-5)
    t_x = params['bn_node_b'] - xst[0] * s_x

    gather = {16: _make_gather(16), 32: _make_gather(32)}
    scatter = _make_scatter()

    x_cur = x
    sx_cur, tx_cur = s_x, t_x
    for i, (nin, nout) in enumerate(DIMS):
        w1 = params['nn1_W%d' % i]
        w1p = s_e[:, None] * w1
        b1p = params['nn1_b%d' % i] + t_e @ w1
        w2 = params['nn2_W%d' % i]
        b2 = params['nn2_b%d' % i]
        rmat = jnp.kron(jnp.eye(nin, dtype=f32), jnp.ones((1, nout), f32))
        smat = jnp.kron(jnp.ones((nin, 1), f32), jnp.eye(nout, dtype=f32))

        xs = gather[nin](x_cur, src3)
        msg = _msg(e_pad, xs, w1p, b1p[None], w2, b2[None], rmat, smat,
                   sx_cur[None], tx_cur[None], nin, nout)
        part = scatter(msg, dst3s, zeros_n32)

        root = params['root%d' % i]
        bias = params['bias%d' % i]
        if i == 0:
            root = s_x[:, None] * params['root0']
            bias = bias + t_x @ params['root0']
        x_cur = _update(part, x_cur, root, bias[None])
        sx_cur = jnp.ones((nout,), f32)
        tx_cur = jnp.zeros((nout,), f32)

    em_w1 = params['em_W1']
    a_w = em_w1[0:32]
    b_w = em_w1[32:64]
    c_w = s_e[:, None] * em_w1[64:]
    b1p = params['em_b1'] + t_e @ em_w1[64:]
    w2p = jnp.zeros((64, 32), f32).at[:, :EDGE_IN].set(params['em_W2'])
    b2p = jnp.zeros((32,), f32).at[:EDGE_IN].set(params['em_b2'])

    xs = gather[32](x_cur, src3)
    xd = gather[32](x_cur, dst3g)
    e_new = _emlp(xs, xd, e_pad, a_w, b_w, c_w, b1p[None], w2p, b2p[None])
    gpart = scatter(e_new, dst3s, zeros_n32)

    nm_w1 = params['nm_W1']
    w1a = nm_w1[0:32]
    w1b = jnp.zeros((32, 64), f32).at[:EDGE_IN, :].set(nm_w1[32:])
    return _head(x_cur, gpart, w1a, w1b, params['nm_b1'][None],
                 params['nm_W2'], params['nm_b2'][None])


# speedup vs baseline: 2.6279x; 1.1684x over previous
import functools

import jax
import jax.numpy as jnp
from jax import lax
from jax.experimental import pallas as pl
from jax.experimental.pallas import tpu as pltpu
from jax.experimental.pallas import tpu_sc as plsc

N = 10000
E = 160000
NODE_IN = 16
EDGE_IN = 19
LEAK = 0.1
DIMS = [(16, 32), (32, 32), (32, 32)]

NC = 2
NS = 16
NW = NC * NS
CHUNK = 128
NCH = 40
EPW = NCH * CHUNK
E_PAD = NW * EPW
PAD = E_PAD - E
N_ACC = N + PAD
RPS = N // NS

ETILE = 2048


def _leaky(v):
    return jnp.where(v >= 0, v, LEAK * v)


def _estats_body(e_ref, o_ref):
    i = pl.program_id(0)
    blk = e_ref[...]
    s = jnp.sum(blk, axis=0, keepdims=True)
    q = jnp.sum(blk * blk, axis=0, keepdims=True)
    part = jnp.concatenate([s, q], axis=0)

    @pl.when(i == 0)
    def _():
        o_ref[...] = part

    @pl.when(i != 0)
    def _():
        o_ref[...] += part


def _edge_stats(e_pad):
    rows = 1280
    grid = E_PAD // rows
    return pl.pallas_call(
        _estats_body,
        grid=(grid,),
        in_specs=[pl.BlockSpec((rows, EDGE_IN), lambda i: (i, 0))],
        out_specs=pl.BlockSpec((2, EDGE_IN), lambda i: (0, 0)),
        out_shape=jax.ShapeDtypeStruct((2, EDGE_IN), jnp.float32),
    )(e_pad)


def _xstats_body(x_ref, o_ref):
    x = x_ref[...]
    mu = jnp.mean(x, axis=0, keepdims=True)
    var = jnp.mean((x - mu) ** 2, axis=0, keepdims=True)
    o_ref[...] = jnp.concatenate([mu, var], axis=0)


def _node_stats(x):
    return pl.pallas_call(
        _xstats_body,
        out_shape=jax.ShapeDtypeStruct((2, NODE_IN), jnp.float32),
    )(x)


def _msg_body(e_ref, xs_ref, w1_ref, b1_ref, w2_ref, b2_ref, r_ref, s_ref,
              sx_ref, tx_ref, o_ref):
    bf = jnp.bfloat16
    xsn = (xs_ref[...] * sx_ref[...] + tx_ref[...]).astype(bf)
    h = _leaky(jnp.dot(e_ref[...], w1_ref[...],
                       preferred_element_type=jnp.float32)
               + b1_ref[...]).astype(bf)
    w = _leaky(jnp.dot(h, w2_ref[...],
                       preferred_element_type=jnp.float32).astype(bf)
               + b2_ref[...])
    xsb = jnp.dot(xsn, r_ref[...],
                  preferred_element_type=jnp.float32).astype(bf)
    o_ref[...] = jnp.dot(xsb * w, s_ref[...],
                         preferred_element_type=jnp.float32)


def _msg(e_pad, xs, w1, b1, w2, b2, rmat, smat, sx, tx, nin, nout):
    grid = E_PAD // ETILE
    full = lambda a: pl.BlockSpec(a.shape, lambda i: tuple(0 for _ in a.shape))
    return pl.pallas_call(
        _msg_body,
        grid=(grid,),
        in_specs=[
            pl.BlockSpec((ETILE, EDGE_IN), lambda i: (i, 0)),
            pl.BlockSpec((ETILE, nin), lambda i: (i, 0)),
            full(w1), full(b1), full(w2), full(b2), full(rmat), full(smat),
            full(sx), full(tx),
        ],
        out_specs=pl.BlockSpec((ETILE, nout), lambda i: (i, 0)),
        out_shape=jax.ShapeDtypeStruct((E_PAD, nout), jnp.float32),
        compiler_params=pltpu.CompilerParams(
            dimension_semantics=("arbitrary",)),
    )(e_pad, xs, w1, b1, w2, b2, rmat, smat, sx, tx)


def _update_body(p_ref, x_ref, rt_ref, bs_ref, o_ref):
    o_ref[...] = (p_ref[0] + p_ref[1]
                  + jnp.dot(x_ref[...], rt_ref[...],
                            preferred_element_type=jnp.float32)
                  + bs_ref[...])


def _update(part, x_cur, root, bias):
    return pl.pallas_call(
        _update_body,
        out_shape=jax.ShapeDtypeStruct((N, root.shape[1]), jnp.float32),
    )(part, x_cur, root, bias)


def _emlp_body(xs_ref, xd_ref, e_ref, a_ref, b_ref, c_ref, b1_ref,
               w2_ref, b2_ref, o_ref):
    bf = jnp.bfloat16
    h = (jnp.dot(xs_ref[...].astype(bf), a_ref[...],
                 preferred_element_type=jnp.float32)
         + jnp.dot(xd_ref[...].astype(bf), b_ref[...],
                   preferred_element_type=jnp.float32)
         + jnp.dot(e_ref[...].astype(bf), c_ref[...],
                   preferred_element_type=jnp.float32)
         + b1_ref[...])
    h = _leaky(h).astype(bf)
    o_ref[...] = jnp.dot(h, w2_ref[...],
                         preferred_element_type=jnp.float32) + b2_ref[...]


def _emlp(xboth, e_pad, a, b, c, b1, w2p, b2p):
    grid = E_PAD // ETILE
    off = E_PAD // ETILE
    full = lambda arr: pl.BlockSpec(arr.shape,
                                    lambda i: tuple(0 for _ in arr.shape))
    return pl.pallas_call(
        _emlp_body,
        grid=(grid,),
        in_specs=[
            pl.BlockSpec((ETILE, 32), lambda i: (i, 0)),
            pl.BlockSpec((ETILE, 32), lambda i: (i + off, 0)),
            pl.BlockSpec((ETILE, EDGE_IN), lambda i: (i, 0)),
            full(a), full(b), full(c), full(b1), full(w2p), full(b2p),
        ],
        out_specs=pl.BlockSpec((ETILE, 32), lambda i: (i, 0)),
        out_shape=jax.ShapeDtypeStruct((E_PAD, 32), jnp.float32),
        compiler_params=pltpu.CompilerParams(
            dimension_semantics=("arbitrary",)),
    )(xboth, xboth, e_pad, a, b, c, b1, w2p, b2p)


def _head_body(x_ref, g_ref, w1a_ref, w1b_ref, b1_ref, w2_ref, b2_ref, o_ref):
    g = g_ref[0] + g_ref[1]
    h = _leaky(jnp.dot(x_ref[...], w1a_ref[...],
                       preferred_element_type=jnp.float32)
               + jnp.dot(g, w1b_ref[...], preferred_element_type=jnp.float32)
               + b1_ref[...])
    v = jnp.dot(h, w2_ref[...], preferred_element_type=jnp.float32) + b2_ref[...]
    m = jnp.max(v, axis=1, keepdims=True)
    lse = m + jnp.log(jnp.sum(jnp.exp(v - m), axis=1, keepdims=True))
    o_ref[...] = v - lse


def _head(x_cur, gpart, w1a, w1b, b1, w2, b2):
    return pl.pallas_call(
        _head_body,
        out_shape=jax.ShapeDtypeStruct((N, 2), jnp.float32),
    )(x_cur, gpart, w1a, w1b, b1, w2, b2)


NBUF = 8


def _sc_gather_body(nch, x_hbm, idx_hbm, out_hbm, idxv, rbuf, *sems):
    gs = sems[:NBUF]
    ss = sems[NBUF:]
    cid = lax.axis_index("c")
    sid = lax.axis_index("s")
    wid = cid * NS + sid
    base = wid * (nch * CHUNK)
    pltpu.sync_copy(idx_hbm.at[wid], idxv)
    for b in range(NBUF):
        pltpu.async_copy(x_hbm.at[idxv.at[b]], rbuf.at[b], gs[b])

    def body(k, carry):
        for b in range(NBUF):
            j = k * NBUF + b
            pltpu.make_async_copy(
                x_hbm.at[idxv.at[0]], rbuf.at[b], gs[b]).wait()
            pltpu.async_copy(
                rbuf.at[b], out_hbm.at[pl.ds(base + j * CHUNK, CHUNK)], ss[b])

            @pl.when(j + NBUF < nch)
            def _(b=b, j=j):
                pltpu.make_async_copy(
                    rbuf.at[b], out_hbm.at[pl.ds(base, CHUNK)], ss[b]).wait()
                pltpu.async_copy(x_hbm.at[idxv.at[j + NBUF]], rbuf.at[b],
                                 gs[b])
        return carry

    lax.fori_loop(0, nch // NBUF, body, 0)
    for b in range(NBUF):
        pltpu.make_async_copy(
            rbuf.at[b], out_hbm.at[pl.ds(base, CHUNK)], ss[b]).wait()


@functools.lru_cache(maxsize=None)
def _make_gather(d, nch=NCH):
    mesh = plsc.VectorSubcoreMesh(core_axis_name="c", subcore_axis_name="s")
    return pl.kernel(
        functools.partial(_sc_gather_body, nch),
        out_type=jax.ShapeDtypeStruct((NW * nch * CHUNK, d), jnp.float32),
        mesh=mesh,
        scratch_types=[
            pltpu.VMEM((nch, CHUNK), jnp.int32),
            pltpu.VMEM((NBUF, CHUNK, d), jnp.float32),
        ] + [pltpu.SemaphoreType.DMA] * (2 * NBUF),
        compiler_params=pltpu.CompilerParams(use_tc_tiling_on_sc=False),
    )


def _sc_scatter_body(msg_hbm, idx_hbm, z_hbm, out_hbm, idxv, mbuf, obuf, accum,
                     *sems):
    gs = sems[:NBUF]
    ss = sems[NBUF:]
    cid = lax.axis_index("c")
    sid = lax.axis_index("s")
    wid = cid * NS + sid
    base = wid * EPW
    pltpu.sync_copy(z_hbm.at[pl.ds(sid * RPS, RPS)], obuf)
    pltpu.sync_copy(obuf, accum.at[pl.ds(sid * RPS, RPS)])
    pltpu.sync_copy(idx_hbm.at[wid], idxv)
    plsc.subcore_barrier()
    for b in range(NBUF):
        pltpu.async_copy(
            msg_hbm.at[pl.ds(base + b * CHUNK, CHUNK)], mbuf.at[b], gs[b])

    def body(k, carry):
        for b in range(NBUF):
            j = k * NBUF + b
            pltpu.make_async_copy(
                msg_hbm.at[pl.ds(base, CHUNK)], mbuf.at[b], gs[b]).wait()
            pltpu.async_copy(mbuf.at[b], accum.at[idxv.at[j]], ss[b],
                             add=True)

            @pl.when(j + NBUF < NCH)
            def _(b=b, j=j):
                pltpu.make_async_copy(
                    mbuf.at[b], accum.at[idxv.at[0]], ss[b]).wait()
                pltpu.async_copy(
                    msg_hbm.at[pl.ds(base + (j + NBUF) * CHUNK, CHUNK)],
                    mbuf.at[b], gs[b])
        return carry

    lax.fori_loop(0, NCH // NBUF, body, 0)
    for b in range(NBUF):
        pltpu.make_async_copy(mbuf.at[b], accum.at[idxv.at[0]], ss[b]).wait()
    plsc.subcore_barrier()
    pltpu.sync_copy(accum.at[pl.ds(sid * RPS, RPS)], obuf)
    pltpu.sync_copy(obuf, out_hbm.at[cid, pl.ds(sid * RPS, RPS)])


@functools.lru_cache(maxsize=None)
def _make_scatter():
    mesh = plsc.VectorSubcoreMesh(core_axis_name="c", subcore_axis_name="s")
    return pl.kernel(
        _sc_scatter_body,
        out_type=jax.ShapeDtypeStruct((NC, N, 32), jnp.float32),
        mesh=mesh,
        scratch_types=[
            pltpu.VMEM((NCH, CHUNK), jnp.int32),
            pltpu.VMEM((NBUF, CHUNK, 32), jnp.float32),
            pltpu.VMEM((RPS, 32), jnp.float32),
            pltpu.VMEM_SHARED((N_ACC, 32), jnp.float32),
        ] + [pltpu.SemaphoreType.DMA] * (2 * NBUF),
        compiler_params=pltpu.CompilerParams(use_tc_tiling_on_sc=False),
    )


def kernel(x, edge_index, e, xbatch, params):
    f32 = jnp.float32
    x = x.astype(f32)
    e = e.astype(f32)
    src = edge_index[0]
    dst = edge_index[1]

    pad_fill = (jnp.arange(PAD, dtype=jnp.int32) * 13) % N
    src3 = jnp.concatenate([src, pad_fill]).reshape(NW, NCH, CHUNK)
    dst3g = jnp.concatenate([dst, pad_fill]).reshape(NW, NCH, CHUNK)
    dst3s = jnp.concatenate(
        [dst, N + jnp.arange(PAD, dtype=jnp.int32)]).reshape(NW, NCH, CHUNK)

    e_pad = jnp.zeros((E_PAD, EDGE_IN), f32).at[:E, :].set(e)
    zeros_n32 = jnp.zeros((N, 32), f32)

    est = _edge_stats(e_pad)
    mu_e = est[0] / E
    var_e = est[1] / E - mu_e * mu_e
    s_e = params['bn_edge_g'] / jnp.sqrt(var_e + 1e-5)
    t_e = params['bn_edge_b'] - mu_e * s_e

    xst = _node_stats(x)
    s_x = params['bn_node_g'] / jnp.sqrt(xst[1] + 1e-5)
    t_x = params['bn_node_b'] - xst[0] * s_x

    gather = {16: _make_gather(16), 32: _make_gather(32)}
    scatter = _make_scatter()

    x_cur = x
    sx_cur, tx_cur = s_x, t_x
    for i, (nin, nout) in enumerate(DIMS):
        w1 = params['nn1_W%d' % i]
        w1p = s_e[:, None] * w1
        b1p = params['nn1_b%d' % i] + t_e @ w1
        w2 = params['nn2_W%d' % i]
        b2 = params['nn2_b%d' % i]
        bf = jnp.bfloat16
        rmat = jnp.kron(jnp.eye(nin, dtype=bf), jnp.ones((1, nout), bf))
        smat = jnp.kron(jnp.ones((nin, 1), bf), jnp.eye(nout, dtype=bf))

        xs = gather[nin](x_cur, src3)
        msg = _msg(e_pad, xs, w1p, b1p[None], w2.astype(bf),
                   b2.astype(bf)[None], rmat, smat,
                   sx_cur[None], tx_cur[None], nin, nout)
        part = scatter(msg, dst3s, zeros_n32)

        root = params['root%d' % i]
        bias = params['bias%d' % i]
        if i == 0:
            root = s_x[:, None] * params['root0']
            bias = bias + t_x @ params['root0']
        x_cur = _update(part, x_cur, root, bias[None])
        sx_cur = jnp.ones((nout,), f32)
        tx_cur = jnp.zeros((nout,), f32)

    em_w1 = params['em_W1']
    a_w = em_w1[0:32]
    b_w = em_w1[32:64]
    c_w = s_e[:, None] * em_w1[64:]
    b1p = params['em_b1'] + t_e @ em_w1[64:]
    w2p = jnp.zeros((64, 32), f32).at[:, :EDGE_IN].set(params['em_W2'])
    b2p = jnp.zeros((32,), f32).at[:EDGE_IN].set(params['em_b2'])

    bf = jnp.bfloat16
    sd3 = jnp.concatenate([src3, dst3g]).reshape(NW, 2 * NCH, CHUNK)
    xboth = _make_gather(32, 2 * NCH)(x_cur, sd3)
    e_new = _emlp(xboth, e_pad, a_w.astype(bf), b_w.astype(bf),
                  c_w.astype(bf), b1p[None], w2p.astype(bf), b2p[None])
    gpart = scatter(e_new, dst3s, zeros_n32)

    nm_w1 = params['nm_W1']
    w1a = nm_w1[0:32]
    w1b = jnp.zeros((32, 64), f32).at[:EDGE_IN, :].set(nm_w1[32:])
    return _head(x_cur, gpart, w1a, w1b, params['nm_b1'][None],
                 params['nm_W2'], params['nm_b2'][None])
